# baseline (speedup 1.0000x reference)
import functools

import jax
import jax.numpy as jnp
from jax import lax
from jax.experimental import pallas as pl
from jax.experimental.pallas import tpu as pltpu
from jax.experimental.pallas import tpu_sc as plsc

_N = 10000
_D = 128
_DW = _D // 2
_E = 320000
_NW = 32
_EPW = _E // _NW
_G = 80
_NG = _EPW // _G


def _mlp_block(x_ref, w_ref, b_ref, o_ref):
    y = jnp.dot(x_ref[...], w_ref[...], preferred_element_type=jnp.float32)
    o_ref[...] = jnp.maximum(y + b_ref[...], 0.0).astype(jnp.bfloat16)


def _transform(x, w, b, bl=2000):
    n, d = x.shape
    return pl.pallas_call(
        _mlp_block,
        grid=(n // bl,),
        in_specs=[
            pl.BlockSpec((bl, d), lambda i: (i, 0)),
            pl.BlockSpec((d, d), lambda i: (0, 0)),
            pl.BlockSpec((1, d), lambda i: (0, 0)),
        ],
        out_specs=pl.BlockSpec((bl, d), lambda i: (i, 0)),
        out_shape=jax.ShapeDtypeStruct((n, d), jnp.bfloat16),
    )(x, w, b.reshape(1, d))


def _edge_values(u_tab, v_tab, src, dst, ev):
    mesh = plsc.VectorSubcoreMesh(core_axis_name="c", subcore_axis_name="s")

    @functools.partial(
        pl.kernel,
        mesh=mesh,
        out_type=jax.ShapeDtypeStruct((_E,), jnp.float32),
        compiler_params=pltpu.CompilerParams(
            needs_layout_passes=False, use_tc_tiling_on_sc=False),
        scratch_types=[
            pltpu.VMEM((_EPW,), jnp.int32),
            pltpu.VMEM((_EPW,), jnp.int32),
            pltpu.VMEM((_EPW,), jnp.float32),
            pltpu.VMEM((_EPW,), jnp.float32),
            pltpu.VMEM((2, _G, _DW), jnp.float32),
            pltpu.VMEM((2, _G, _DW), jnp.float32),
            pltpu.SemaphoreType.DMA,
            pltpu.SemaphoreType.DMA,
            pltpu.SemaphoreType.DMA,
            pltpu.SemaphoreType.DMA,
        ],
    )
    def body(u_hbm, v_hbm, src_hbm, dst_hbm, ev_hbm, out_hbm,
             src_v, dst_v, ev_v, out_v, u_rows, v_rows,
             su0, sv0, su1, sv1):
        wid = lax.axis_index("s") * 2 + lax.axis_index("c")
        base = wid * _EPW
        pltpu.sync_copy(src_hbm.at[pl.ds(base, _EPW)], src_v)
        pltpu.sync_copy(dst_hbm.at[pl.ds(base, _EPW)], dst_v)
        pltpu.sync_copy(ev_hbm.at[pl.ds(base, _EPW)], ev_v)

        lanes = lax.iota(jnp.int32, 16)
        sems = ((su0, sv0), (su1, sv1))

        def issue(g, b):
            gb = g * _G
            pltpu.async_copy(u_hbm.at[src_v.at[pl.ds(gb, _G)]],
                             u_rows.at[b], sems[b][0])
            pltpu.async_copy(v_hbm.at[dst_v.at[pl.ds(gb, _G)]],
                             v_rows.at[b], sems[b][1])

        def compute(g, b):
            gb = g * _G
            pltpu.make_async_copy(u_hbm.at[src_v.at[pl.ds(gb, _G)]],
                                  u_rows.at[b], sems[b][0]).wait()
            pltpu.make_async_copy(v_hbm.at[dst_v.at[pl.ds(gb, _G)]],
                                  v_rows.at[b], sems[b][1]).wait()

            def subgroup(sg, c):
                sgb = sg * 16
                d2 = jnp.zeros((16,), jnp.float32)
                for k in range(16):
                    e = sgb + k
                    acc = jnp.zeros((16,), jnp.float32)
                    for j in range(_DW // 16):
                        uw = u_rows[b, e, pl.ds(j * 16, 16)]
                        vw = v_rows[b, e, pl.ds(j * 16, 16)]
                        ub = plsc.bitcast(uw, jnp.bfloat16)
                        vb = plsc.bitcast(vw, jnp.bfloat16)
                        du = ub - vb
                        lo, hi = plsc.unpack(
                            du, format=plsc.PackFormat.INTERLEAVED)
                        acc = acc + lo * lo + hi * hi
                    d2 = jnp.where(lanes == k, jnp.sum(acc), d2)
                d2c = jnp.maximum(d2, 1e-30)
                bi = lax.bitcast_convert_type(d2c, jnp.int32)
                bi = 0x5F3759DF - lax.shift_right_arithmetic(bi, 1)
                y = lax.bitcast_convert_type(bi, jnp.float32)
                for _ in range(3):
                    y = y * (1.5 - 0.5 * d2c * y * y)
                dist = d2 * y
                sim = jnp.exp(dist)
                sig = 1.0 / (1.0 + jnp.exp(-sim))
                eb = gb + sgb
                out_v[pl.ds(eb, 16)] = ev_v[pl.ds(eb, 16)] * sig
                return c

            lax.fori_loop(0, _G // 16, subgroup, 0)

        issue(0, 0)

        def outer(tt, carry):
            g0 = tt * 2
            issue(g0 + 1, 1)
            compute(g0, 0)
            issue(g0 + 2, 0)
            compute(g0 + 1, 1)
            return carry

        lax.fori_loop(0, (_NG - 1) // 2, outer, 0)
        compute(_NG - 1, 0)
        pltpu.sync_copy(out_v, out_hbm.at[pl.ds(base, _EPW)])

    return body(u_tab, v_tab, src, dst, ev)


def kernel(Eu, Ev, W1, b1, W2, b2, edge_index, edge_val):
    u = _transform(Eu, W1, b1)
    v = _transform(Ev, W2, b2)
    u_pk = lax.bitcast_convert_type(u.reshape(_N, _DW, 2), jnp.float32)
    v_pk = lax.bitcast_convert_type(v.reshape(_N, _DW, 2), jnp.float32)
    return _edge_values(u_pk, v_pk, edge_index[0], edge_index[1], edge_val)


# speedup vs baseline: 1.0211x; 1.0211x over previous
import functools

import jax
import jax.numpy as jnp
from jax import lax
from jax.experimental import pallas as pl
from jax.experimental.pallas import tpu as pltpu
from jax.experimental.pallas import tpu_sc as plsc

_N = 10000
_D = 128
_DW = _D // 2
_E = 320000
_NW = 32
_EPW = _E // _NW
_G = 80
_NG = _EPW // _G


def _mlp_block(x_ref, w_ref, b_ref, o_ref):
    y = jnp.dot(x_ref[...], w_ref[...], preferred_element_type=jnp.float32)
    o_ref[...] = jnp.maximum(y + b_ref[...], 0.0).astype(jnp.bfloat16)


def _transform(x, w, b, bl=2000):
    n, d = x.shape
    return pl.pallas_call(
        _mlp_block,
        grid=(n // bl,),
        in_specs=[
            pl.BlockSpec((bl, d), lambda i: (i, 0)),
            pl.BlockSpec((d, d), lambda i: (0, 0)),
            pl.BlockSpec((1, d), lambda i: (0, 0)),
        ],
        out_specs=pl.BlockSpec((bl, d), lambda i: (i, 0)),
        out_shape=jax.ShapeDtypeStruct((n, d), jnp.bfloat16),
    )(x, w, b.reshape(1, d))


def _edge_values(u_tab, v_tab, src, dst, ev):
    mesh = plsc.VectorSubcoreMesh(core_axis_name="c", subcore_axis_name="s")

    @functools.partial(
        pl.kernel,
        mesh=mesh,
        out_type=jax.ShapeDtypeStruct((_E,), jnp.float32),
        compiler_params=pltpu.CompilerParams(
            needs_layout_passes=False, use_tc_tiling_on_sc=False),
        scratch_types=[
            pltpu.VMEM((_EPW,), jnp.int32),
            pltpu.VMEM((_EPW,), jnp.int32),
            pltpu.VMEM((_EPW,), jnp.float32),
            pltpu.VMEM((_EPW,), jnp.float32),
            pltpu.VMEM((2, _G, _DW), jnp.float32),
            pltpu.VMEM((2, _G, _DW), jnp.float32),
            pltpu.VMEM_SHARED((_N, _DW), jnp.float32),
            pltpu.SemaphoreType.DMA,
            pltpu.SemaphoreType.DMA,
            pltpu.SemaphoreType.DMA,
            pltpu.SemaphoreType.DMA,
        ],
    )
    def body(u_hbm, v_hbm, src_hbm, dst_hbm, ev_hbm, out_hbm,
             src_v, dst_v, ev_v, out_v, u_rows, v_rows, u_sh,
             su0, sv0, su1, sv1):
        wid = lax.axis_index("s") * 2 + lax.axis_index("c")
        base = wid * _EPW

        @pl.when(lax.axis_index("s") == 0)
        def _stage_tables():
            pltpu.sync_copy(u_hbm, u_sh)

        pltpu.sync_copy(src_hbm.at[pl.ds(base, _EPW)], src_v)
        pltpu.sync_copy(dst_hbm.at[pl.ds(base, _EPW)], dst_v)
        pltpu.sync_copy(ev_hbm.at[pl.ds(base, _EPW)], ev_v)
        plsc.subcore_barrier()

        lanes = lax.iota(jnp.int32, 16)
        sems = ((su0, sv0), (su1, sv1))

        def issue(g, b):
            gb = g * _G
            pltpu.async_copy(u_sh.at[src_v.at[pl.ds(gb, _G)]],
                             u_rows.at[b], sems[b][0])
            pltpu.async_copy(v_hbm.at[dst_v.at[pl.ds(gb, _G)]],
                             v_rows.at[b], sems[b][1])

        def compute(g, b):
            gb = g * _G
            pltpu.make_async_copy(u_sh.at[src_v.at[pl.ds(gb, _G)]],
                                  u_rows.at[b], sems[b][0]).wait()
            pltpu.make_async_copy(v_hbm.at[dst_v.at[pl.ds(gb, _G)]],
                                  v_rows.at[b], sems[b][1]).wait()

            def subgroup(sg, c):
                sgb = sg * 16
                d2 = jnp.zeros((16,), jnp.float32)
                for k in range(16):
                    e = sgb + k
                    acc = jnp.zeros((16,), jnp.float32)
                    for j in range(_DW // 16):
                        uw = u_rows[b, e, pl.ds(j * 16, 16)]
                        vw = v_rows[b, e, pl.ds(j * 16, 16)]
                        ub = plsc.bitcast(uw, jnp.bfloat16)
                        vb = plsc.bitcast(vw, jnp.bfloat16)
                        du = ub - vb
                        lo, hi = plsc.unpack(
                            du, format=plsc.PackFormat.INTERLEAVED)
                        acc = acc + lo * lo + hi * hi
                    d2 = jnp.where(lanes == k, jnp.sum(acc), d2)
                d2c = jnp.maximum(d2, 1e-30)
                bi = lax.bitcast_convert_type(d2c, jnp.int32)
                bi = 0x5F3759DF - lax.shift_right_arithmetic(bi, 1)
                y = lax.bitcast_convert_type(bi, jnp.float32)
                for _ in range(3):
                    y = y * (1.5 - 0.5 * d2c * y * y)
                dist = d2 * y
                sim = jnp.exp(dist)
                sig = 1.0 / (1.0 + jnp.exp(-sim))
                eb = gb + sgb
                out_v[pl.ds(eb, 16)] = ev_v[pl.ds(eb, 16)] * sig
                return c

            lax.fori_loop(0, _G // 16, subgroup, 0)

        issue(0, 0)

        def outer(tt, carry):
            g0 = tt * 2
            issue(g0 + 1, 1)
            compute(g0, 0)
            issue(g0 + 2, 0)
            compute(g0 + 1, 1)
            return carry

        lax.fori_loop(0, (_NG - 1) // 2, outer, 0)
        compute(_NG - 1, 0)
        pltpu.sync_copy(out_v, out_hbm.at[pl.ds(base, _EPW)])

    return body(u_tab, v_tab, src, dst, ev)


def kernel(Eu, Ev, W1, b1, W2, b2, edge_index, edge_val):
    u = _transform(Eu, W1, b1)
    v = _transform(Ev, W2, b2)
    u_pk = lax.bitcast_convert_type(u.reshape(_N, _DW, 2), jnp.float32)
    v_pk = lax.bitcast_convert_type(v.reshape(_N, _DW, 2), jnp.float32)
    return _edge_values(u_pk, v_pk, edge_index[0], edge_index[1], edge_val)


# speedup vs baseline: 1.0648x; 1.0429x over previous
import functools

import jax
import jax.numpy as jnp
from jax import lax
from jax.experimental import pallas as pl
from jax.experimental.pallas import tpu as pltpu
from jax.experimental.pallas import tpu_sc as plsc

_N = 10000
_D = 128
_DW = _D // 2
_E = 320000
_NW = 32
_EPW = _E // _NW
_G = 80
_NG = _EPW // _G


def _mlp_block(x_ref, w_ref, b_ref, o_ref):
    y = jnp.dot(x_ref[...], w_ref[...], preferred_element_type=jnp.float32)
    o_ref[...] = jnp.maximum(y + b_ref[...], 0.0).astype(jnp.bfloat16)


def _transform(x, w, b, bl=2000):
    n, d = x.shape
    return pl.pallas_call(
        _mlp_block,
        grid=(n // bl,),
        in_specs=[
            pl.BlockSpec((bl, d), lambda i: (i, 0)),
            pl.BlockSpec((d, d), lambda i: (0, 0)),
            pl.BlockSpec((1, d), lambda i: (0, 0)),
        ],
        out_specs=pl.BlockSpec((bl, d), lambda i: (i, 0)),
        out_shape=jax.ShapeDtypeStruct((n, d), jnp.bfloat16),
    )(x, w, b.reshape(1, d))


def _edge_values(u_tab, v_tab, src, dst, ev):
    mesh = plsc.VectorSubcoreMesh(core_axis_name="c", subcore_axis_name="s")

    @functools.partial(
        pl.kernel,
        mesh=mesh,
        out_type=jax.ShapeDtypeStruct((_E,), jnp.float32),
        compiler_params=pltpu.CompilerParams(
            needs_layout_passes=False, use_tc_tiling_on_sc=False),
        scratch_types=[
            pltpu.VMEM((_EPW,), jnp.int32),
            pltpu.VMEM((_EPW,), jnp.int32),
            pltpu.VMEM((_EPW,), jnp.float32),
            pltpu.VMEM((_EPW,), jnp.float32),
            pltpu.VMEM((3, _G, _DW), jnp.float32),
            pltpu.VMEM((3, _G, _DW), jnp.float32),
            pltpu.SemaphoreType.DMA,
            pltpu.SemaphoreType.DMA,
            pltpu.SemaphoreType.DMA,
            pltpu.SemaphoreType.DMA,
            pltpu.SemaphoreType.DMA,
            pltpu.SemaphoreType.DMA,
        ],
    )
    def body(u_hbm, v_hbm, src_hbm, dst_hbm, ev_hbm, out_hbm,
             src_v, dst_v, ev_v, out_v, u_rows, v_rows,
             su0, sv0, su1, sv1, su2, sv2):
        wid = lax.axis_index("s") * 2 + lax.axis_index("c")
        base = wid * _EPW
        pltpu.sync_copy(src_hbm.at[pl.ds(base, _EPW)], src_v)
        pltpu.sync_copy(dst_hbm.at[pl.ds(base, _EPW)], dst_v)
        pltpu.sync_copy(ev_hbm.at[pl.ds(base, _EPW)], ev_v)

        lanes = lax.iota(jnp.int32, 16)
        sems = ((su0, sv0), (su1, sv1), (su2, sv2))

        def issue(g, b):
            gb = g * _G
            pltpu.async_copy(u_hbm.at[src_v.at[pl.ds(gb, _G)]],
                             u_rows.at[b], sems[b][0])
            pltpu.async_copy(v_hbm.at[dst_v.at[pl.ds(gb, _G)]],
                             v_rows.at[b], sems[b][1])

        def compute(g, b):
            gb = g * _G
            pltpu.make_async_copy(u_hbm.at[src_v.at[pl.ds(gb, _G)]],
                                  u_rows.at[b], sems[b][0]).wait()
            pltpu.make_async_copy(v_hbm.at[dst_v.at[pl.ds(gb, _G)]],
                                  v_rows.at[b], sems[b][1]).wait()

            def subgroup(sg, c):
                sgb = sg * 16
                d2 = jnp.zeros((16,), jnp.float32)
                for k in range(16):
                    e = sgb + k
                    acc = jnp.zeros((16,), jnp.float32)
                    for j in range(_DW // 16):
                        uw = u_rows[b, e, pl.ds(j * 16, 16)]
                        vw = v_rows[b, e, pl.ds(j * 16, 16)]
                        ub = plsc.bitcast(uw, jnp.bfloat16)
                        vb = plsc.bitcast(vw, jnp.bfloat16)
                        du = ub - vb
                        lo, hi = plsc.unpack(
                            du, format=plsc.PackFormat.INTERLEAVED)
                        acc = acc + lo * lo + hi * hi
                    d2 = jnp.where(lanes == k, jnp.sum(acc), d2)
                d2c = jnp.maximum(d2, 1e-30)
                bi = lax.bitcast_convert_type(d2c, jnp.int32)
                bi = 0x5F3759DF - lax.shift_right_arithmetic(bi, 1)
                y = lax.bitcast_convert_type(bi, jnp.float32)
                for _ in range(3):
                    y = y * (1.5 - 0.5 * d2c * y * y)
                dist = d2 * y
                sim = jnp.exp(dist)
                sig = 1.0 / (1.0 + jnp.exp(-sim))
                eb = gb + sgb
                out_v[pl.ds(eb, 16)] = ev_v[pl.ds(eb, 16)] * sig
                return c

            lax.fori_loop(0, _G // 16, subgroup, 0)

        issue(0, 0)
        issue(1, 1)

        def outer(tt, carry):
            g0 = tt * 3
            for k in range(3):
                issue(g0 + k + 2, (k + 2) % 3)
                compute(g0 + k, k)
            return carry

        lax.fori_loop(0, (_NG - 2) // 3, outer, 0)
        compute(_NG - 2, 0)
        compute(_NG - 1, 1)
        pltpu.sync_copy(out_v, out_hbm.at[pl.ds(base, _EPW)])

    return body(u_tab, v_tab, src, dst, ev)


def kernel(Eu, Ev, W1, b1, W2, b2, edge_index, edge_val):
    u = _transform(Eu, W1, b1)
    v = _transform(Ev, W2, b2)
    u_pk = lax.bitcast_convert_type(u.reshape(_N, _DW, 2), jnp.float32)
    v_pk = lax.bitcast_convert_type(v.reshape(_N, _DW, 2), jnp.float32)
    return _edge_values(u_pk, v_pk, edge_index[0], edge_index[1], edge_val)
